# baseline (speedup 1.0000x reference)
import functools

import jax
import jax.numpy as jnp
from jax import lax
from jax.experimental import pallas as pl
from jax.experimental.pallas import tpu as pltpu
from jax.experimental.pallas import tpu_sc as plsc


def _tc_pack(table, n_cols):
    v, d = table.shape
    pairs = v // (2 * n_cols)
    off = pairs * n_cols
    tail_start = 2 * off
    steps = pairs + 1
    f32 = jnp.float32

    tail = jax.lax.slice(table, (tail_start, 0), (v, d))
    tail = jnp.pad(tail, ((0, n_cols - (v - tail_start)), (0, 0)))

    def body(lo_ref, hi_ref, tail_ref, out_ref):
        step = pl.program_id(0)

        @pl.when(step < pairs)
        def _pair():
            out_ref[...] = jnp.concatenate(
                [lo_ref[...].T, hi_ref[...].T], axis=1)

        @pl.when(step == pairs)
        def _tail():
            out_ref[...] = jnp.concatenate(
                [tail_ref[...], jnp.zeros_like(tail_ref[...])], axis=1)

    packed = pl.pallas_call(
        body,
        grid=(steps,),
        in_specs=[
            pl.BlockSpec((d, n_cols), lambda l: (0, jnp.minimum(l, pairs - 1))),
            pl.BlockSpec(
                (d, n_cols),
                lambda l: (0, jnp.minimum(l + pairs, 2 * pairs - 1))),
            pl.BlockSpec((n_cols, d), lambda l: (0, 0)),
        ],
        out_specs=pl.BlockSpec((n_cols, 2 * d), lambda l: (l, 0)),
        out_shape=jax.ShapeDtypeStruct((steps * n_cols, 2 * d), f32),
    )(table.T, table.T, tail)
    return packed, off, tail_start


def _sc_info():
    info = plsc.get_sparse_core_info()
    return info.num_cores, info.num_subcores


def _sc_gather_hist(hist_ridx, item_ridx, itab_rows, n_hist, n_side, d):
    nc, ns = _sc_info()
    nw = nc * ns
    lanes = 128
    rows_w = n_hist // nw
    streams_w = rows_w // lanes
    s_per_chunk = 10
    chunks = streams_w // s_per_chunk
    chunk_rows = s_per_chunk * lanes
    side_w = n_side // nw // lanes

    mesh = plsc.VectorSubcoreMesh(core_axis_name="c", subcore_axis_name="s")
    f32 = jnp.float32

    @functools.partial(
        pl.kernel,
        out_type=(
            jax.ShapeDtypeStruct((n_hist, d), f32),
            jax.ShapeDtypeStruct((n_side, d), f32),
        ),
        mesh=mesh,
        compiler_params=pltpu.CompilerParams(use_tc_tiling_on_sc=False),
        scratch_types=[
            pltpu.VMEM((streams_w, lanes), jnp.int32),
            pltpu.VMEM((chunk_rows, d), f32),
            pltpu.VMEM((side_w, lanes), jnp.int32),
            pltpu.VMEM((lanes, d), f32),
            pltpu.SemaphoreType.DMA,
        ],
    )
    def gather_kernel(hist_idx_h, item_idx_h, itab_h,
                      hist_out, item_out,
                      idx_v, rows_v, sidx_v, srows_v, sem):
        wid = lax.axis_index("s") * nc + lax.axis_index("c")

        pltpu.sync_copy(item_idx_h.at[wid], sidx_v)
        for j in range(side_w):
            pltpu.async_copy(itab_h.at[sidx_v.at[j]], srows_v, sem).wait()
            pltpu.sync_copy(
                srows_v, item_out.at[pl.ds((wid * side_w + j) * lanes, lanes)])

        pltpu.sync_copy(hist_idx_h.at[wid], idx_v)

        @pl.loop(0, chunks)
        def _chunk(c):
            descs = [
                pltpu.async_copy(
                    itab_h.at[idx_v.at[c * s_per_chunk + jj]],
                    rows_v.at[pl.ds(jj * lanes, lanes)],
                    sem,
                )
                for jj in range(s_per_chunk)
            ]
            for desc in descs:
                desc.wait()
            pltpu.sync_copy(
                rows_v,
                hist_out.at[pl.ds(wid * rows_w + c * chunk_rows, chunk_rows)],
            )

    return gather_kernel(
        hist_ridx.reshape(nw, streams_w, lanes),
        item_ridx.reshape(nw, side_w, lanes),
        itab_rows)


def _sc_gather_user(user_ridx, utab_rows, n_side, d):
    nc, ns = _sc_info()
    nw = nc * ns
    lanes = 128
    side_w = n_side // nw // lanes

    mesh = plsc.VectorSubcoreMesh(core_axis_name="c", subcore_axis_name="s")

    @functools.partial(
        pl.kernel,
        out_type=jax.ShapeDtypeStruct((n_side, d), jnp.float32),
        mesh=mesh,
        compiler_params=pltpu.CompilerParams(use_tc_tiling_on_sc=False),
        scratch_types=[
            pltpu.VMEM((side_w, lanes), jnp.int32),
            pltpu.VMEM((lanes, d), jnp.float32),
            pltpu.SemaphoreType.DMA,
        ],
    )
    def gather_kernel(user_idx_h, utab_h, user_out, sidx_v, srows_v, sem):
        wid = lax.axis_index("s") * nc + lax.axis_index("c")
        pltpu.sync_copy(user_idx_h.at[wid], sidx_v)
        for j in range(side_w):
            pltpu.async_copy(utab_h.at[sidx_v.at[j]], srows_v, sem).wait()
            pltpu.sync_copy(
                srows_v, user_out.at[pl.ds((wid * side_w + j) * lanes, lanes)])

    return gather_kernel(user_ridx.reshape(nw, side_w, lanes), utab_rows)


def _tc_dense(hist3, user_emb, item_emb,
              w_int, w1u_t, w1h_t, b1r, w2r, b2r, d):
    n_l, n_bh, _ = hist3.shape
    n_b = 2 * n_bh
    lps = 2 if n_l % 2 == 0 else 1
    f32 = jnp.float32

    def body(hist_ref, u_ref, it_ref,
             wint_ref, w1u_ref, w1h_ref, b1_ref, w2_ref, b2_ref,
             uout_ref, inter_ref, upart_s):
        step = pl.program_id(0)

        @pl.when(step == 0)
        def _init():
            u = u_ref[...]
            upart_s[...] = (
                jnp.dot(u, w1u_ref[...], preferred_element_type=f32)
                + b1_ref[...]
            )
            t = jnp.dot(u, wint_ref[...], preferred_element_type=f32)
            inter_ref[...] = jnp.sum(t * it_ref[...], axis=1, keepdims=True)
            uout_ref[...] = u

        hists = [
            jnp.concatenate(
                [hist_ref[s][:, :d], hist_ref[s][:, d:]], axis=0)
            for s in range(lps)
        ]
        hh = jnp.concatenate(hists, axis=0) if lps > 1 else hists[0]
        mm = jnp.dot(hh.astype(jnp.bfloat16),
                     w1h_ref[...].astype(jnp.bfloat16),
                     preferred_element_type=f32)
        up = upart_s[...]
        if lps > 1:
            up = jnp.concatenate([up] * lps, axis=0)
        h = jnp.maximum(mm + up, 0.0)
        a = jax.nn.sigmoid(
            jnp.dot(h.astype(jnp.bfloat16),
                    w2_ref[...].astype(jnp.bfloat16).T,
                    preferred_element_type=f32) + b2_ref[0, 0]
        )
        c = a * hh
        acc = c[:n_b]
        for s in range(1, lps):
            acc = acc + c[s * n_b:(s + 1) * n_b]
        uout_ref[...] += acc

    full = lambda shape: pl.BlockSpec(shape, lambda l: (0,) * len(shape))
    return pl.pallas_call(
        body,
        grid=(n_l // lps,),
        in_specs=[
            pl.BlockSpec((lps, n_bh, 2 * d), lambda l: (l, 0, 0)),
            full((n_b, d)),
            full((n_b, d)),
            full((d, d)),
            full((d, d)),
            full((d, d)),
            full((1, d)),
            full((1, d)),
            full((1, 1)),
        ],
        out_specs=[full((n_b, d)), full((n_b, 1))],
        out_shape=[
            jax.ShapeDtypeStruct((n_b, d), f32),
            jax.ShapeDtypeStruct((n_b, 1), f32),
        ],
        scratch_shapes=[pltpu.VMEM((n_b, d), f32)],
    )(hist3, user_emb, item_emb, w_int, w1u_t, w1h_t, b1r, w2r, b2r)


def kernel(user_ids, item_ids, user_history, user_table, item_table,
           W_int, w1, b1, w2, b2):
    n_b, n_l = user_history.shape
    d = user_table.shape[1]
    h_dim = w1.shape[0]

    itab2, off, tail_start = _tc_pack(item_table, 8192)
    utab2, _, _ = _tc_pack(user_table, 8192)
    itab_rows = itab2.reshape(-1, d)
    utab_rows = utab2.reshape(-1, d)

    def map_ids(i):
        return jnp.where(
            i >= tail_start, 2 * (off + i - tail_start),
            jnp.where(i >= off, 2 * (i - off) + 1, 2 * i))

    idx_t = user_history.T
    ilv = jnp.stack(
        [idx_t[:, :n_b // 2], idx_t[:, n_b // 2:]], axis=2).reshape(-1)
    hist_ridx = map_ids(ilv)
    hist_rows, item_emb = _sc_gather_hist(
        hist_ridx, map_ids(item_ids), itab_rows, n_l * n_b, n_b, d)
    user_emb = _sc_gather_user(map_ids(user_ids), utab_rows, n_b, d)

    user_out, interaction = _tc_dense(
        hist_rows.reshape(n_l, n_b // 2, 2 * d), user_emb, item_emb,
        W_int, w1[:, :d].T, w1[:, d:].T,
        b1.reshape(1, h_dim), w2.reshape(1, h_dim),
        b2.reshape(1, 1).astype(jnp.float32), d)

    return (user_out, item_emb, interaction)


# speedup vs baseline: 1.3858x; 1.3858x over previous
import functools

import jax
import jax.numpy as jnp
from jax import lax
from jax.experimental import pallas as pl
from jax.experimental.pallas import tpu as pltpu
from jax.experimental.pallas import tpu_sc as plsc


def _tc_pack(table, n_cols):
    v, d = table.shape
    pairs = v // (2 * n_cols)
    off = pairs * n_cols
    tail_start = 2 * off
    steps = pairs + 1
    f32 = jnp.float32

    tail = jax.lax.slice(table, (tail_start, 0), (v, d))
    tail = jnp.pad(tail, ((0, n_cols - (v - tail_start)), (0, 0)))

    def body(lo_ref, hi_ref, tail_ref, out_ref):
        step = pl.program_id(0)

        @pl.when(step < pairs)
        def _pair():
            out_ref[...] = jnp.concatenate(
                [lo_ref[...].T, hi_ref[...].T], axis=1)

        @pl.when(step == pairs)
        def _tail():
            out_ref[...] = jnp.concatenate(
                [tail_ref[...], jnp.zeros_like(tail_ref[...])], axis=1)

    packed = pl.pallas_call(
        body,
        grid=(steps,),
        in_specs=[
            pl.BlockSpec((d, n_cols), lambda l: (0, jnp.minimum(l, pairs - 1))),
            pl.BlockSpec(
                (d, n_cols),
                lambda l: (0, jnp.minimum(l + pairs, 2 * pairs - 1))),
            pl.BlockSpec((n_cols, d), lambda l: (0, 0)),
        ],
        out_specs=pl.BlockSpec((n_cols, 2 * d), lambda l: (l, 0)),
        out_shape=jax.ShapeDtypeStruct((steps * n_cols, 2 * d), f32),
    )(table.T, table.T, tail)
    return packed, off, tail_start


def _tc_hist_idx(user_history, off, tail_start):
    n_b, n_l = user_history.shape

    def body(in_ref, out_ref):
        i = in_ref[...]
        m = jnp.where(
            i >= tail_start, 2 * (off + i - tail_start),
            jnp.where(i >= off, 2 * (i - off) + 1, 2 * i))
        out_ref[...] = m.T

    return pl.pallas_call(
        body,
        in_specs=[pl.BlockSpec((n_b, n_l), lambda: (0, 0))],
        out_specs=pl.BlockSpec((n_l, n_b), lambda: (0, 0)),
        out_shape=jax.ShapeDtypeStruct((n_l, n_b), jnp.int32),
    )(user_history)


def _sc_info():
    info = plsc.get_sparse_core_info()
    return info.num_cores, info.num_subcores


def _sc_gather_hist(hist_ridx, item_ridx, itab_rows, n_l, n_b, n_side, d):
    nc, ns = _sc_info()
    nw = nc * ns
    lanes = 128
    s_per_chunk = 10
    chunks = n_l // s_per_chunk
    chunk_rows = s_per_chunk * lanes
    side_w = n_side // nw // lanes

    mesh = plsc.VectorSubcoreMesh(core_axis_name="c", subcore_axis_name="s")
    f32 = jnp.float32

    @functools.partial(
        pl.kernel,
        out_type=(
            jax.ShapeDtypeStruct((n_l * n_b, d), f32),
            jax.ShapeDtypeStruct((n_side, d), f32),
        ),
        mesh=mesh,
        compiler_params=pltpu.CompilerParams(use_tc_tiling_on_sc=False),
        scratch_types=[
            pltpu.VMEM((n_l, lanes), jnp.int32),
            pltpu.VMEM((chunk_rows, d), f32),
            pltpu.VMEM((side_w, lanes), jnp.int32),
            pltpu.VMEM((lanes, d), f32),
            pltpu.SemaphoreType.DMA,
        ],
    )
    def gather_kernel(hist_idx_h, item_idx_h, itab_h,
                      hist_out, item_out,
                      idx_v, rows_v, sidx_v, srows_v, sem):
        wid = lax.axis_index("s") * nc + lax.axis_index("c")

        pltpu.sync_copy(item_idx_h.at[wid], sidx_v)
        for j in range(side_w):
            pltpu.async_copy(itab_h.at[sidx_v.at[j]], srows_v, sem).wait()
            pltpu.sync_copy(
                srows_v, item_out.at[pl.ds((wid * side_w + j) * lanes, lanes)])

        pltpu.sync_copy(hist_idx_h.at[:, pl.ds(wid * lanes, lanes)], idx_v)

        @pl.loop(0, chunks)
        def _chunk(c):
            descs = [
                pltpu.async_copy(
                    itab_h.at[idx_v.at[c * s_per_chunk + jj]],
                    rows_v.at[pl.ds(jj * lanes, lanes)],
                    sem,
                )
                for jj in range(s_per_chunk)
            ]
            for desc in descs:
                desc.wait()
            for jj in range(s_per_chunk):
                pltpu.sync_copy(
                    rows_v.at[pl.ds(jj * lanes, lanes)],
                    hist_out.at[pl.ds(
                        (c * s_per_chunk + jj) * n_b + wid * lanes, lanes)],
                )

    return gather_kernel(hist_ridx, item_ridx.reshape(nw, side_w, lanes),
                         itab_rows)


def _sc_gather_user(user_ridx, utab_rows, n_side, d):
    nc, ns = _sc_info()
    nw = nc * ns
    lanes = 128
    side_w = n_side // nw // lanes

    mesh = plsc.VectorSubcoreMesh(core_axis_name="c", subcore_axis_name="s")

    @functools.partial(
        pl.kernel,
        out_type=jax.ShapeDtypeStruct((n_side, d), jnp.float32),
        mesh=mesh,
        compiler_params=pltpu.CompilerParams(use_tc_tiling_on_sc=False),
        scratch_types=[
            pltpu.VMEM((side_w, lanes), jnp.int32),
            pltpu.VMEM((lanes, d), jnp.float32),
            pltpu.SemaphoreType.DMA,
        ],
    )
    def gather_kernel(user_idx_h, utab_h, user_out, sidx_v, srows_v, sem):
        wid = lax.axis_index("s") * nc + lax.axis_index("c")
        pltpu.sync_copy(user_idx_h.at[wid], sidx_v)
        for j in range(side_w):
            pltpu.async_copy(utab_h.at[sidx_v.at[j]], srows_v, sem).wait()
            pltpu.sync_copy(
                srows_v, user_out.at[pl.ds((wid * side_w + j) * lanes, lanes)])

    return gather_kernel(user_ridx.reshape(nw, side_w, lanes), utab_rows)


def _tc_dense(hist3, user_emb, item_emb,
              w_int, w1u_t, w1h_t, b1r, w2r, b2r, d):
    n_l, n_bh, _ = hist3.shape
    n_b = 2 * n_bh
    lps = 2 if n_l % 2 == 0 else 1
    f32 = jnp.float32

    def body(hist_ref, u_ref, it_ref,
             wint_ref, w1u_ref, w1h_ref, b1_ref, w2_ref, b2_ref,
             uout_ref, inter_ref, upart_s):
        step = pl.program_id(0)

        @pl.when(step == 0)
        def _init():
            u = u_ref[...]
            upart_s[...] = (
                jnp.dot(u, w1u_ref[...], preferred_element_type=f32)
                + b1_ref[...]
            )
            t = jnp.dot(u, wint_ref[...], preferred_element_type=f32)
            inter_ref[...] = jnp.sum(t * it_ref[...], axis=1, keepdims=True)
            uout_ref[...] = u

        hists = [
            jnp.concatenate(
                [hist_ref[s][:, :d], hist_ref[s][:, d:]], axis=0)
            for s in range(lps)
        ]
        hh = jnp.concatenate(hists, axis=0) if lps > 1 else hists[0]
        mm = jnp.dot(hh.astype(jnp.bfloat16),
                     w1h_ref[...].astype(jnp.bfloat16),
                     preferred_element_type=f32)
        up = upart_s[...]
        if lps > 1:
            up = jnp.concatenate([up] * lps, axis=0)
        h = jnp.maximum(mm + up, 0.0)
        a = jax.nn.sigmoid(
            jnp.dot(h.astype(jnp.bfloat16),
                    w2_ref[...].astype(jnp.bfloat16).T,
                    preferred_element_type=f32) + b2_ref[0, 0]
        )
        c = a * hh
        acc = c[:n_b]
        for s in range(1, lps):
            acc = acc + c[s * n_b:(s + 1) * n_b]
        uout_ref[...] += acc

    full = lambda shape: pl.BlockSpec(shape, lambda l: (0,) * len(shape))
    return pl.pallas_call(
        body,
        grid=(n_l // lps,),
        in_specs=[
            pl.BlockSpec((lps, n_bh, 2 * d), lambda l: (l, 0, 0)),
            full((n_b, d)),
            full((n_b, d)),
            full((d, d)),
            full((d, d)),
            full((d, d)),
            full((1, d)),
            full((1, d)),
            full((1, 1)),
        ],
        out_specs=[full((n_b, d)), full((n_b, 1))],
        out_shape=[
            jax.ShapeDtypeStruct((n_b, d), f32),
            jax.ShapeDtypeStruct((n_b, 1), f32),
        ],
        scratch_shapes=[pltpu.VMEM((n_b, d), f32)],
    )(hist3, user_emb, item_emb, w_int, w1u_t, w1h_t, b1r, w2r, b2r)


def kernel(user_ids, item_ids, user_history, user_table, item_table,
           W_int, w1, b1, w2, b2):
    n_b, n_l = user_history.shape
    d = user_table.shape[1]
    h_dim = w1.shape[0]

    itab2, off, tail_start = _tc_pack(item_table, 8192)
    utab2, _, _ = _tc_pack(user_table, 8192)
    itab_rows = itab2.reshape(-1, d)
    utab_rows = utab2.reshape(-1, d)

    def map_ids(i):
        return jnp.where(
            i >= tail_start, 2 * (off + i - tail_start),
            jnp.where(i >= off, 2 * (i - off) + 1, 2 * i))

    def perm(x):
        return jnp.concatenate([x[0::2], x[1::2]])

    def unperm(x):
        return jnp.stack(
            [x[:n_b // 2], x[n_b // 2:]], axis=1).reshape(n_b, -1)

    hist_ridx = _tc_hist_idx(user_history, off, tail_start)
    hist_rows, item_p = _sc_gather_hist(
        hist_ridx, map_ids(perm(item_ids)), itab_rows, n_l, n_b, n_b, d)
    user_p = _sc_gather_user(map_ids(perm(user_ids)), utab_rows, n_b, d)

    uout_p, inter_p = _tc_dense(
        hist_rows.reshape(n_l, n_b // 2, 2 * d), user_p, item_p,
        W_int, w1[:, :d].T, w1[:, d:].T,
        b1.reshape(1, h_dim), w2.reshape(1, h_dim),
        b2.reshape(1, 1).astype(jnp.float32), d)

    return (unperm(uout_p), unperm(item_p), unperm(inter_p))
